# baseline (speedup 1.0000x reference)
import functools

import jax
import jax.numpy as jnp
from jax import lax
from jax.experimental import pallas as pl
from jax.experimental.pallas import tpu as pltpu
from jax.experimental.pallas import tpu_sc as plsc

NC = 2
NS = 16
LB = 128
DW = 16


def _tc_z(x, W1, W2):
    n, f = x.shape
    nh = W1.shape[1]
    ncls = W2.shape[1]
    nb = 5
    br = n // nb

    def body(x_ref, w1_ref, w2_ref, z_ref):
        wc = jnp.dot(w1_ref[...], w2_ref[...], preferred_element_type=jnp.float32)
        z_ref[...] = jnp.dot(x_ref[...], wc, preferred_element_type=jnp.float32)

    return pl.pallas_call(
        body,
        grid=(nb,),
        in_specs=[
            pl.BlockSpec((br, f), lambda i: (i, 0)),
            pl.BlockSpec((f, nh), lambda i: (0, 0)),
            pl.BlockSpec((nh, ncls), lambda i: (0, 0)),
        ],
        out_specs=pl.BlockSpec((br, ncls), lambda i: (i, 0)),
        out_shape=jax.ShapeDtypeStruct((n, ncls), jnp.float32),
    )(x, W1, W2)


def _rsqrt16(x):
    i = plsc.bitcast(x, jnp.int32)
    i = jnp.int32(0x5F3759DF) - lax.shift_right_logical(i, 1)
    y = plsc.bitcast(i, jnp.float32)
    for _ in range(3):
        y = y * (1.5 - 0.5 * x * y * y)
    return y


def _make_sc_kernel(n, n_pad, rt):
    npw = n_pad // NS
    fpw = 2 * npw // NC
    nw_full = (2 * n) // fpw
    rem = 2 * n - nw_full * fpw

    @functools.partial(
        pl.kernel,
        out_type=jax.ShapeDtypeStruct((2 * n,), jnp.float32),
        mesh=plsc.VectorSubcoreMesh(core_axis_name="c", subcore_axis_name="s"),
        scratch_types=[
            pltpu.VMEM((rt, LB), jnp.int32),
            pltpu.VMEM((rt, LB), jnp.int32),
            pltpu.VMEM((2 * n_pad,), jnp.float32),
            pltpu.VMEM((n_pad,), jnp.float32),
            pltpu.VMEM((n_pad,), jnp.float32),
            pltpu.VMEM((npw,), jnp.float32),
            pltpu.VMEM((npw,), jnp.float32),
            pltpu.VMEM((fpw // 2,), jnp.float32),
            pltpu.VMEM((fpw // 2,), jnp.float32),
            pltpu.VMEM((fpw,), jnp.float32),
            pltpu.VMEM((LB,), jnp.float32),
            pltpu.VMEM((8, LB), jnp.float32),
            pltpu.VMEM((128,), jnp.float32),
            pltpu.VMEM((256,), jnp.float32),
            pltpu.VMEM((8,), jnp.float32),
            pltpu.VMEM_SHARED((n_pad,), jnp.float32),
            pltpu.VMEM_SHARED((n_pad,), jnp.float32),
            pltpu.VMEM_SHARED((n_pad,), jnp.float32),
            pltpu.VMEM_SHARED((n_pad,), jnp.float32),
            pltpu.SemaphoreType.DMA,
            pltpu.SemaphoreType.DMA,
        ],
        compiler_params=pltpu.CompilerParams(needs_layout_passes=False),
    )
    def sc_kernel(src_hbm, dst_hbm, zf_hbm, dinit_hbm, zeros_hbm, ones_hbm,
                  b1_hbm, w2_hbm, b2_hbm, out_hbm,
                  src_v, dst_v, z_v, dis_v, zi_v, degs_v, diss_v,
                  a0s_v, a1s_v, out_v, ones_v, m_v,
                  b1_v, w2_v, b2_v, deg_sh, dis_sh, acc0_sh, acc1_sh,
                  sem_in, sem):
        c = lax.axis_index("c")
        s = lax.axis_index("s")
        w = c * NS + s
        lanes = lax.iota(jnp.int32, 16)

        cps = [
            pltpu.async_copy(src_hbm.at[s], src_v, sem_in),
            pltpu.async_copy(dst_hbm.at[s], dst_v, sem_in),
            pltpu.async_copy(zf_hbm, z_v.at[pl.ds(0, 2 * n)], sem_in),
            pltpu.async_copy(ones_hbm, ones_v, sem_in),
            pltpu.async_copy(b1_hbm, b1_v, sem_in),
            pltpu.async_copy(w2_hbm, w2_v, sem_in),
            pltpu.async_copy(b2_hbm, b2_v, sem_in),
        ]

        @pl.when(s == 0)
        def _():
            pltpu.sync_copy(dinit_hbm, zi_v)
            pltpu.sync_copy(zi_v, deg_sh)
            pltpu.sync_copy(zeros_hbm, zi_v)
            pltpu.sync_copy(zi_v, acc0_sh)
            pltpu.sync_copy(zi_v, acc1_sh)

        for cp in cps:
            cp.wait()
        plsc.subcore_barrier()

        def deg_body(j, carry):
            @pl.when(j >= DW)
            def _():
                pltpu.make_async_copy(
                    ones_v, deg_sh.at[dst_v.at[j - DW]], sem).wait()
            pltpu.async_copy(ones_v, deg_sh.at[dst_v.at[j]], sem, add=True)
            return carry

        lax.fori_loop(0, rt, deg_body, 0, unroll=False)
        for j in range(max(rt - DW, 0), rt):
            pltpu.make_async_copy(ones_v, deg_sh.at[dst_v.at[j]], sem).wait()
        plsc.subcore_barrier()

        pltpu.sync_copy(deg_sh.at[pl.ds(npw * s, npw)], degs_v)

        def dis_body(g, carry):
            degv = degs_v[pl.ds(g * 16, 16)]
            diss_v[pl.ds(g * 16, 16)] = _rsqrt16(degv)
            return carry

        lax.fori_loop(0, npw // 16, dis_body, 0, unroll=False)
        pltpu.sync_copy(diss_v, dis_sh.at[pl.ds(npw * s, npw)])
        plsc.subcore_barrier()
        pltpu.sync_copy(dis_sh, dis_v)

        def gather_fire(j, b):
            for k in range(LB // 16):
                s16 = src_v[j, pl.ds(k * 16, 16)]
                fi = s16 * 2
                g0 = plsc.load_gather(z_v, [fi])
                g1 = plsc.load_gather(z_v, [fi + 1])
                dv = plsc.load_gather(dis_v, [s16])
                m_v[2 * b, pl.ds(k * 16, 16)] = g0 * dv
                m_v[2 * b + 1, pl.ds(k * 16, 16)] = g1 * dv
            pltpu.async_copy(m_v.at[2 * b], acc0_sh.at[dst_v.at[j]],
                             sem, add=True)
            pltpu.async_copy(m_v.at[2 * b + 1], acc1_sh.at[dst_v.at[j]],
                             sem, add=True)

        def drain(j, b):
            pltpu.make_async_copy(m_v.at[2 * b],
                                  acc0_sh.at[dst_v.at[j]], sem).wait()
            pltpu.make_async_copy(m_v.at[2 * b + 1],
                                  acc1_sh.at[dst_v.at[j]], sem).wait()

        def edge_body(t, carry):
            @pl.when(t >= 1)
            def _():
                for b in range(4):
                    drain(4 * t + b - 4, b)

            for b in range(4):
                gather_fire(4 * t + b, b)
            return carry

        lax.fori_loop(0, rt // 4, edge_body, 0, unroll=False)
        for b in range(4):
            drain(rt - 4 + b, b)
        plsc.subcore_barrier()

        def brow_vec(col):
            p = jnp.zeros((16,), jnp.float32)
            for k in range(8):
                bv = b1_v[pl.ds(k * 16, 16)]
                wv = plsc.load_gather(w2_v, [(lanes + k * 16) * 2 + col])
                p = p + bv * wv
            b2v = plsc.load_gather(b2_v, [jnp.full((16,), col, jnp.int32)])
            return jnp.broadcast_to(jnp.sum(p), (16,)) + b2v

        brow0 = brow_vec(0)
        brow1 = brow_vec(1)

        node_lo = (fpw // 2) * w
        pltpu.sync_copy(acc0_sh.at[pl.ds(node_lo, fpw // 2)], a0s_v)
        pltpu.sync_copy(acc1_sh.at[pl.ds(node_lo, fpw // 2)], a1s_v)

        def fin_body(g, carry):
            base = g * 16
            a0 = a0s_v[pl.ds(base, 16)]
            a1 = a1s_v[pl.ds(base, 16)]
            d = dis_v[pl.ds(node_lo + base, 16)]
            node16 = node_lo + base + lanes
            z0 = plsc.load_gather(z_v, [node16 * 2])
            z1 = plsc.load_gather(z_v, [node16 * 2 + 1])
            o0 = d * (a0 + z0 * d) + brow0
            o1 = d * (a1 + z1 * d) + brow1
            oi = (base + lanes) * 2
            plsc.store_scatter(out_v, [oi], o0)
            plsc.store_scatter(out_v, [oi + 1], o1)
            return carry

        lax.fori_loop(0, fpw // 2 // 16, fin_body, 0, unroll=False)

        @pl.when(w < nw_full)
        def _():
            pltpu.sync_copy(out_v, out_hbm.at[pl.ds(2 * node_lo, fpw)])

        if rem > 0:
            @pl.when(w == nw_full)
            def _():
                pltpu.sync_copy(out_v.at[pl.ds(0, rem)],
                                out_hbm.at[pl.ds(2 * node_lo, rem)])

    return sc_kernel


def kernel(x, edge_index, W1, b1, W2, b2):
    n, f = x.shape
    e = edge_index.shape[1]
    ncls = W2.shape[1]

    n_pad = ((n + 8 + 2047) // 2048) * 2048
    rt = (e + NS * LB - 1) // (NS * LB)
    rt = ((rt + 3) // 4) * 4
    e_pad = NS * rt * LB

    src = edge_index[0]
    dst = edge_index[1]
    pad_idx = jnp.full((e_pad - e,), n, dtype=jnp.int32)
    src_p = jnp.concatenate([src, pad_idx]).reshape(NS, rt, LB)
    dst_p = jnp.concatenate([dst, pad_idx]).reshape(NS, rt, LB)
    dinit = jnp.pad(jnp.ones((n,), jnp.float32), (0, n_pad - n))
    zeros1 = jnp.zeros((n_pad,), jnp.float32)
    ones_lb = jnp.ones((LB,), jnp.float32)

    z = _tc_z(x, W1, W2)
    outf = _make_sc_kernel(n, n_pad, rt)(
        src_p, dst_p, z.reshape(-1), dinit, zeros1, ones_lb,
        b1, W2.reshape(-1), jnp.pad(b2, (0, 8 - ncls)))
    return outf.reshape(n, ncls)


# speedup vs baseline: 1.0572x; 1.0572x over previous
import functools

import jax
import jax.numpy as jnp
from jax import lax
from jax.experimental import pallas as pl
from jax.experimental.pallas import tpu as pltpu
from jax.experimental.pallas import tpu_sc as plsc

NC = 2
NS = 16
LB = 128
DW = 8


def _tc_z(x, W1, W2):
    n, f = x.shape
    nh = W1.shape[1]
    ncls = W2.shape[1]
    nb = 5
    br = n // nb

    def body(x_ref, w1_ref, w2_ref, z_ref):
        wc = jnp.dot(w1_ref[...], w2_ref[...], preferred_element_type=jnp.float32)
        z_ref[...] = jnp.dot(x_ref[...], wc, preferred_element_type=jnp.float32)

    return pl.pallas_call(
        body,
        grid=(nb,),
        in_specs=[
            pl.BlockSpec((br, f), lambda i: (i, 0)),
            pl.BlockSpec((f, nh), lambda i: (0, 0)),
            pl.BlockSpec((nh, ncls), lambda i: (0, 0)),
        ],
        out_specs=pl.BlockSpec((br, ncls), lambda i: (i, 0)),
        out_shape=jax.ShapeDtypeStruct((n, ncls), jnp.float32),
    )(x, W1, W2)


def _rsqrt16(x):
    i = plsc.bitcast(x, jnp.int32)
    i = jnp.int32(0x5F3759DF) - lax.shift_right_logical(i, 1)
    y = plsc.bitcast(i, jnp.float32)
    for _ in range(3):
        y = y * (1.5 - 0.5 * x * y * y)
    return y


def _make_sc_kernel(n, n_pad, rt):
    npw = n_pad // NS
    fpw = 2 * npw // NC
    nw_full = (2 * n) // fpw
    rem = 2 * n - nw_full * fpw

    @functools.partial(
        pl.kernel,
        out_type=jax.ShapeDtypeStruct((2 * n,), jnp.float32),
        mesh=plsc.VectorSubcoreMesh(core_axis_name="c", subcore_axis_name="s"),
        scratch_types=[
            pltpu.VMEM((rt, LB), jnp.int32),
            pltpu.VMEM((rt, LB), jnp.int32),
            pltpu.VMEM((2 * n_pad,), jnp.float32),
            pltpu.VMEM((n_pad,), jnp.float32),
            pltpu.VMEM((2 * n_pad,), jnp.float32),
            pltpu.VMEM((NS, fpw), jnp.float32),
            pltpu.VMEM((fpw,), jnp.float32),
            pltpu.VMEM((npw,), jnp.float32),
            pltpu.VMEM((npw,), jnp.float32),
            pltpu.VMEM((fpw,), jnp.float32),
            pltpu.VMEM((LB,), jnp.float32),
            pltpu.VMEM((128,), jnp.float32),
            pltpu.VMEM((256,), jnp.float32),
            pltpu.VMEM((8,), jnp.float32),
            pltpu.VMEM_SHARED((n_pad,), jnp.float32),
            pltpu.VMEM_SHARED((n_pad,), jnp.float32),
            pltpu.VMEM_SHARED((NS, 2 * n_pad), jnp.float32),
            pltpu.SemaphoreType.DMA,
            pltpu.SemaphoreType.DMA,
        ],
        compiler_params=pltpu.CompilerParams(needs_layout_passes=False),
    )
    def sc_kernel(src_hbm, dst_hbm, zf_hbm, dinit_hbm, zeros_hbm, ones_hbm,
                  b1_hbm, w2_hbm, b2_hbm, out_hbm,
                  src_v, dst_v, z_v, dis_v, acc_v, mb_v, asum_v,
                  degs_v, diss_v, out_v, ones_v,
                  b1_v, w2_v, b2_v, deg_sh, dis_sh, acc_all,
                  sem_in, sem):
        c = lax.axis_index("c")
        s = lax.axis_index("s")
        w = c * NS + s
        lanes = lax.iota(jnp.int32, 16)

        cps = [
            pltpu.async_copy(src_hbm.at[s], src_v, sem_in),
            pltpu.async_copy(dst_hbm.at[s], dst_v, sem_in),
            pltpu.async_copy(zf_hbm, z_v.at[pl.ds(0, 2 * n)], sem_in),
            pltpu.async_copy(ones_hbm, ones_v, sem_in),
            pltpu.async_copy(b1_hbm, b1_v, sem_in),
            pltpu.async_copy(w2_hbm, w2_v, sem_in),
            pltpu.async_copy(b2_hbm, b2_v, sem_in),
        ]

        @pl.when(s == 0)
        def _():
            pltpu.sync_copy(dinit_hbm, acc_v.at[pl.ds(0, n_pad)])
            pltpu.sync_copy(acc_v.at[pl.ds(0, n_pad)], deg_sh)

        pltpu.sync_copy(zeros_hbm, acc_v.at[pl.ds(0, n_pad)])
        pltpu.sync_copy(zeros_hbm, acc_v.at[pl.ds(n_pad, n_pad)])

        for cp in cps:
            cp.wait()
        plsc.subcore_barrier()

        def deg_body(j, carry):
            @pl.when(j >= DW)
            def _():
                pltpu.make_async_copy(
                    ones_v, deg_sh.at[dst_v.at[j - DW]], sem).wait()
            pltpu.async_copy(ones_v, deg_sh.at[dst_v.at[j]], sem, add=True)
            return carry

        lax.fori_loop(0, rt, deg_body, 0, unroll=False)
        for j in range(max(rt - DW, 0), rt):
            pltpu.make_async_copy(ones_v, deg_sh.at[dst_v.at[j]], sem).wait()
        plsc.subcore_barrier()

        pltpu.sync_copy(deg_sh.at[pl.ds(npw * s, npw)], degs_v)

        def dis_body(g, carry):
            degv = degs_v[pl.ds(g * 16, 16)]
            diss_v[pl.ds(g * 16, 16)] = _rsqrt16(degv)
            return carry

        lax.fori_loop(0, npw // 16, dis_body, 0, unroll=False)
        pltpu.sync_copy(diss_v, dis_sh.at[pl.ds(npw * s, npw)])
        plsc.subcore_barrier()
        pltpu.sync_copy(dis_sh, dis_v)

        def edge_body(j, carry):
            for k in range(LB // 16):
                s16 = src_v[j, pl.ds(k * 16, 16)]
                d16 = dst_v[j, pl.ds(k * 16, 16)]
                fi = s16 * 2
                di = d16 * 2
                g0 = plsc.load_gather(z_v, [fi])
                g1 = plsc.load_gather(z_v, [fi + 1])
                dv = plsc.load_gather(dis_v, [s16])
                plsc.addupdate_scatter(acc_v, [di], g0 * dv)
                plsc.addupdate_scatter(acc_v, [di + 1], g1 * dv)
            return carry

        lax.fori_loop(0, rt, edge_body, 0, unroll=False)
        pltpu.sync_copy(acc_v, acc_all.at[s])
        plsc.subcore_barrier()

        def brow_vec(col):
            p = jnp.zeros((16,), jnp.float32)
            for k in range(8):
                bv = b1_v[pl.ds(k * 16, 16)]
                wv = plsc.load_gather(w2_v, [(lanes + k * 16) * 2 + col])
                p = p + bv * wv
            b2v = plsc.load_gather(b2_v, [jnp.full((16,), col, jnp.int32)])
            return jnp.broadcast_to(jnp.sum(p), (16,)) + b2v

        brow0 = brow_vec(0)
        brow1 = brow_vec(1)

        node_lo = (fpw // 2) * w
        pltpu.sync_copy(acc_all.at[:, pl.ds(2 * node_lo, fpw)], mb_v)

        def sum_body(g, carry):
            base = g * 16
            acc16 = mb_v[0, pl.ds(base, 16)]
            for q in range(1, NS):
                acc16 = acc16 + mb_v[q, pl.ds(base, 16)]
            asum_v[pl.ds(base, 16)] = acc16
            return carry

        lax.fori_loop(0, fpw // 16, sum_body, 0, unroll=False)

        def fin_body(g, carry):
            base = g * 16
            bl2 = (base + lanes) * 2
            a0 = plsc.load_gather(asum_v, [bl2])
            a1 = plsc.load_gather(asum_v, [bl2 + 1])
            d = dis_v[pl.ds(node_lo + base, 16)]
            node16 = node_lo + base + lanes
            z0 = plsc.load_gather(z_v, [node16 * 2])
            z1 = plsc.load_gather(z_v, [node16 * 2 + 1])
            o0 = d * (a0 + z0 * d) + brow0
            o1 = d * (a1 + z1 * d) + brow1
            oi = (base + lanes) * 2
            plsc.store_scatter(out_v, [oi], o0)
            plsc.store_scatter(out_v, [oi + 1], o1)
            return carry

        lax.fori_loop(0, fpw // 2 // 16, fin_body, 0, unroll=False)

        @pl.when(w < nw_full)
        def _():
            pltpu.sync_copy(out_v, out_hbm.at[pl.ds(2 * node_lo, fpw)])

        if rem > 0:
            @pl.when(w == nw_full)
            def _():
                pltpu.sync_copy(out_v.at[pl.ds(0, rem)],
                                out_hbm.at[pl.ds(2 * node_lo, rem)])

    return sc_kernel


def kernel(x, edge_index, W1, b1, W2, b2):
    n, f = x.shape
    e = edge_index.shape[1]
    ncls = W2.shape[1]

    n_pad = ((n + 8 + 2047) // 2048) * 2048
    rt = (e + NS * LB - 1) // (NS * LB)
    rt = rt + (rt & 1)
    e_pad = NS * rt * LB

    src = edge_index[0]
    dst = edge_index[1]
    pad_idx = jnp.full((e_pad - e,), n, dtype=jnp.int32)
    src_p = jnp.concatenate([src, pad_idx]).reshape(NS, rt, LB)
    dst_p = jnp.concatenate([dst, pad_idx]).reshape(NS, rt, LB)
    dinit = jnp.pad(jnp.ones((n,), jnp.float32), (0, n_pad - n))
    zeros1 = jnp.zeros((n_pad,), jnp.float32)
    ones_lb = jnp.ones((LB,), jnp.float32)

    z = _tc_z(x, W1, W2)
    outf = _make_sc_kernel(n, n_pad, rt)(
        src_p, dst_p, z.reshape(-1), dinit, zeros1, ones_lb,
        b1, W2.reshape(-1), jnp.pad(b2, (0, 8 - ncls)))
    return outf.reshape(n, ncls)
